# baseline (speedup 1.0000x reference)
import functools

import jax
import jax.numpy as jnp
from jax import lax
from jax.experimental import pallas as pl
from jax.experimental.pallas import tpu as pltpu
from jax.experimental.pallas import tpu_sc as plsc

_N = 50000
_E = 800000
_INF = 4
_H = 128
_BOUT = 4096

_NTILE = 32
_IX = 128
_CH = 512
_NSUB = _CH // _IX
_EPAD = 819200
_EPT_A = _EPAD // 32
_EPT_B = _EPAD // 16
_R = 50176
_ZROWS = _R // 16

def _mesh():
    return plsc.VectorSubcoreMesh(core_axis_name="c", subcore_axis_name="s")


def _zero_stage(rows):
    def zr(i, c2):
        r = i // 2
        l = (i % 2) * 16
        rows[r, pl.ds(l, 16)] = jnp.zeros((16,), jnp.float32)
        return c2

    lax.fori_loop(0, 1024, zr, 0)


def _zero_acc(acc, rows, s):
    _zero_stage(rows)
    zoff = pl.multiple_of(s * _ZROWS, 8)
    for k in range(7):
        pltpu.sync_copy(rows.at[pl.ds(0, 448)],
                        acc.at[pl.ds(zoff + k * 448, 448)])


def _flush_acc(acc, rows, s, out_h, out_base):
    zoff = pl.multiple_of(s * _ZROWS, 8)
    obase = pl.multiple_of(out_base + s * _ZROWS, 8)
    for k in range(7):
        pltpu.sync_copy(acc.at[pl.ds(zoff + k * 448, 448)],
                        rows.at[pl.ds(0, 448)])
        pltpu.sync_copy(rows.at[pl.ds(0, 448)],
                        out_h.at[pl.ds(obase + k * 448, 448)])


def _fire_idx_loads(src_h, dst_h, iv, dv, rb, sem):
    for j in range(_NSUB):
        pltpu.async_copy(src_h.at[rb + j], iv[j], sem)
        pltpu.async_copy(dst_h.at[rb + j], dv[j], sem)


def _drain_idx_loads(src_h, dst_h, iv, dv, sem):
    for j in range(_NSUB):
        pltpu.make_async_copy(src_h.at[0], iv[j], sem).wait()
        pltpu.make_async_copy(dst_h.at[0], dv[j], sem).wait()


def _agg_pass(feat_h, src_h, dst_h, acc, iv, dv, rows, seml, gsems, erow0,
              nch, qoff, raw=None):
    iva, ivb = iv[:_NSUB], iv[_NSUB:]
    dva, dvb = dv[:_NSUB], dv[_NSUB:]
    maxrb = erow0 + (nch - 1) * _NSUB

    def half(ck, ivx, dvx, ivn, dvn):
        _drain_idx_loads(src_h, dst_h, ivx, dvx, seml)
        rbn = jnp.minimum(erow0 + (ck + 1) * _NSUB, maxrb)
        _fire_idx_loads(src_h, dst_h, ivn, dvn, rbn, seml)
        if qoff is not None:
            for j in range(_NSUB):
                for l in range(_IX // 16):
                    ivx[j][pl.ds(l * 16, 16)] = (
                        ivx[j][pl.ds(l * 16, 16)] + qoff)
        gdst = rows if raw is None else raw
        handles = [
            pltpu.async_copy(feat_h.at[ivx[j]],
                             gdst.at[pl.ds(j * _IX, _IX)], gsems[j])
            for j in range(_NSUB)
        ]
        for j in range(_NSUB):
            handles[j].wait()
            if raw is not None:
                def unp(r, c3):
                    jr = j * _IX + r
                    w = raw[jr, pl.ds(0, 16)]
                    a = jax.lax.bitcast_convert_type(
                        jax.lax.shift_left(w, 16), jnp.float32)
                    b = jax.lax.bitcast_convert_type(
                        jax.lax.bitwise_and(w, jnp.int32(-65536)),
                        jnp.float32)
                    rows[jr, pl.ds(0, 16)] = a
                    rows[jr, pl.ds(16, 16)] = b
                    return c3
                lax.fori_loop(0, _IX, unp, 0)
            pltpu.sync_copy(rows.at[pl.ds(j * _IX, _IX)],
                            acc.at[dvx[j]], add=True)

    _fire_idx_loads(src_h, dst_h, iva, dva, erow0, seml)

    def pair(k, carry):
        half(2 * k, iva, dva, ivb, dvb)
        half(2 * k + 1, ivb, dvb, iva, dva)
        return carry

    lax.fori_loop(0, nch // 2, pair, 0)
    _drain_idx_loads(src_h, dst_h, iva, dva, seml)


def _sc_layer0(xpad_h, src_h, dst_h, out_h, acc,
               d0, d1, d2, d3, d4, d5, d6, d7,
               i0, i1, i2, i3, i4, i5, i6, i7,
               rows, seml, g0, g1, g2, g3):
    c = lax.axis_index("c")
    s = lax.axis_index("s")
    wid = c * 16 + s
    _zero_acc(acc, rows, s)
    plsc.subcore_barrier()
    _agg_pass(xpad_h, src_h, dst_h, acc,
              [i0, i1, i2, i3, i4, i5, i6, i7],
              [d0, d1, d2, d3, d4, d5, d6, d7],
              rows, seml, [g0, g1, g2, g3],
              wid * (_EPT_A // _IX), _EPT_A // _CH, None)
    plsc.subcore_barrier()
    _flush_acc(acc, rows, s, out_h, c * _R)


def _sc_agg(feat_h, src_h, dst_h, out_h, acc,
            d0, d1, d2, d3, d4, d5, d6, d7,
            i0, i1, i2, i3, i4, i5, i6, i7,
            rows, raw, seml, g0, g1, g2, g3):
    c = lax.axis_index("c")
    s = lax.axis_index("s")

    for p in range(2):
        q = c * 2 + p
        qoff = q * _N
        _zero_acc(acc, rows, s)
        plsc.subcore_barrier()
        _agg_pass(feat_h, src_h, dst_h, acc,
                  [i0, i1, i2, i3, i4, i5, i6, i7],
                  [d0, d1, d2, d3, d4, d5, d6, d7],
                  rows, seml, [g0, g1, g2, g3],
                  s * (_EPT_B // _IX), _EPT_B // _CH, qoff, raw=raw)
        plsc.subcore_barrier()
        _flush_acc(acc, rows, s, out_h, q * _R)
        plsc.subcore_barrier()


def _idx_scratch():
    return [pltpu.VMEM((_IX,), jnp.int32) for _ in range(4 * _NSUB)]


def _make_layer0():
    return pl.kernel(
        _sc_layer0,
        out_type=jax.ShapeDtypeStruct((2 * _R, 32), jnp.float32),
        mesh=_mesh(),
        compiler_params=pltpu.CompilerParams(use_tc_tiling_on_sc=False),
        scratch_types=[
            pltpu.VMEM_SHARED((_R, 32), jnp.float32),
        ] + _idx_scratch() + [
            pltpu.VMEM((_CH, 32), jnp.float32),
            pltpu.SemaphoreType.DMA,
            pltpu.SemaphoreType.DMA,
            pltpu.SemaphoreType.DMA,
            pltpu.SemaphoreType.DMA,
            pltpu.SemaphoreType.DMA,
        ],
    )


def _make_agg():
    return pl.kernel(
        _sc_agg,
        out_type=jax.ShapeDtypeStruct((4 * _R, 32), jnp.float32),
        mesh=_mesh(),
        compiler_params=pltpu.CompilerParams(use_tc_tiling_on_sc=False),
        scratch_types=[
            pltpu.VMEM_SHARED((_R, 32), jnp.float32),
        ] + _idx_scratch() + [
            pltpu.VMEM((_CH, 32), jnp.float32),
            pltpu.VMEM((_CH, 16), jnp.int32),
            pltpu.SemaphoreType.DMA,
            pltpu.SemaphoreType.DMA,
            pltpu.SemaphoreType.DMA,
            pltpu.SemaphoreType.DMA,
            pltpu.SemaphoreType.DMA,
        ],
    )


_RB = 2000


def _prelu(v, a):
    return jnp.where(v >= 0, v, a * v)


def _t1_body(part_ref, x_ref, w1_ref, wl0_ref, wr0_ref, b0_ref, a0_ref,
             featq_ref, x1_ref, xth_ref, rec_ref):
    pr = part_ref[...]
    s0 = pr[0, :, 0:4] + pr[1, :, 0:4]
    deg = pr[0, :, 4:5] + pr[1, :, 4:5]
    rec = 1.0 / jnp.maximum(deg, 1.0)
    x = x_ref[...]
    mean0 = s0 * rec
    x1 = _prelu(
        jax.lax.dot_general(mean0, wl0_ref[...], (((1,), (0,)), ((), ())),
                            preferred_element_type=jnp.float32)
        + jax.lax.dot_general(x, wr0_ref[...], (((1,), (0,)), ((), ())),
                              preferred_element_type=jnp.float32)
        + b0_ref[...][None, :],
        a0_ref[...][None, :])
    xth = jax.lax.dot_general(x, w1_ref[...], (((1,), (1,)), ((), ())),
                              preferred_element_type=jnp.float32)
    feat1 = x1 + xth
    for qq in range(4):
        featq_ref[qq] = feat1[:, 32 * qq:32 * (qq + 1)].astype(jnp.bfloat16)
    x1_ref[...] = x1
    xth_ref[...] = xth
    rec_ref[...] = jnp.broadcast_to(rec, (rec.shape[0], 8))


def _t2_body(s1q_ref, x1_ref, xth_ref, rec_ref, wl1_ref, wr1_ref,
             b1_ref, a1_ref, feat2q_ref):
    s1 = jnp.concatenate([s1q_ref[qq] for qq in range(4)], axis=1)
    feat1 = x1_ref[...] + xth_ref[...]
    rec = rec_ref[...][:, 0:1]
    mean1 = s1 * rec
    x2 = _prelu(
        jax.lax.dot_general(mean1, wl1_ref[...], (((1,), (0,)), ((), ())),
                            preferred_element_type=jnp.float32)
        + jax.lax.dot_general(feat1, wr1_ref[...], (((1,), (0,)), ((), ())),
                              preferred_element_type=jnp.float32)
        + b1_ref[...][None, :],
        a1_ref[...][None, :])
    h2 = x1_ref[...] + x2
    feat2 = h2 + xth_ref[...]
    for qq in range(4):
        feat2q_ref[qq] = feat2[:, 32 * qq:32 * (qq + 1)].astype(jnp.bfloat16)


def _t3_body(s2q_ref, feat2q_ref, rec_ref, wl2_ref, wr2_ref, b2_ref, out_ref):
    s2 = jnp.concatenate([s2q_ref[qq] for qq in range(4)], axis=1)
    feat2 = jnp.concatenate([feat2q_ref[qq] for qq in range(4)],
                            axis=1).astype(jnp.float32)
    rec = rec_ref[...][:, 0:1]
    mean2 = s2 * rec
    out_ref[...] = (
        jax.lax.dot_general(mean2, wl2_ref[...], (((1,), (0,)), ((), ())),
                            preferred_element_type=jnp.float32)
        + jax.lax.dot_general(feat2, wr2_ref[...], (((1,), (0,)), ((), ())),
                              preferred_element_type=jnp.float32)
        + b2_ref[...][None, :])


def _full(shape):
    return pl.BlockSpec(shape, lambda i: tuple(0 for _ in shape))


def _col_perm():
    p = []
    for q in range(4):
        for i in range(16):
            p.extend([32 * q + i, 32 * q + 16 + i])
    perm = [0] * 128
    for j, orig in enumerate(p):
        perm[j] = orig
    return jnp.asarray(perm, jnp.int32)


def kernel(x, edge_index, batch_size, W1, Wl0, Wr0, b0, Wl1, Wr1, b1, Wl2,
           Wr2, b2, a0, a1):
    f32 = jnp.float32
    x = x.astype(f32)
    P = _col_perm()
    W1 = W1[P, :]
    Wl0, Wr0, b0, a0 = Wl0[:, P], Wr0[:, P], b0[P], a0[P]
    Wl1, Wr1, b1, a1 = Wl1[:, P], Wr1[P, :][:, P], b1[P], a1[P]
    Wr2 = Wr2[P, :]
    src = edge_index[0].astype(jnp.int32)
    dst = edge_index[1].astype(jnp.int32)
    npad = _EPAD - _E
    src_p = jnp.concatenate([src, jnp.zeros((npad,), jnp.int32)])
    dst_p = jnp.concatenate([dst, jnp.full((npad,), _N, jnp.int32)])
    src2 = src_p.reshape(_EPAD // _IX, _IX)
    dst2 = dst_p.reshape(_EPAD // _IX, _IX)

    xpad = jnp.concatenate(
        [x, jnp.ones((_N, 1), f32), jnp.zeros((_N, 27), f32)], axis=1)

    part = _make_layer0()(xpad, src2, dst2).reshape(2, _R, 32)

    grid = _N // _RB
    t1 = pl.pallas_call(
        _t1_body,
        grid=(grid,),
        in_specs=[
            pl.BlockSpec((2, _RB, 32), lambda i: (0, i, 0)),
            pl.BlockSpec((_RB, 4), lambda i: (i, 0)),
            _full((_H, _INF)),
            _full((_INF, _H)),
            _full((_INF, _H)),
            _full((_H,)),
            _full((_H,)),
        ],
        out_specs=[
            pl.BlockSpec((4, _RB, 32), lambda i: (0, i, 0)),
            pl.BlockSpec((_RB, _H), lambda i: (i, 0)),
            pl.BlockSpec((_RB, _H), lambda i: (i, 0)),
            pl.BlockSpec((_RB, 8), lambda i: (i, 0)),
        ],
        out_shape=[
            jax.ShapeDtypeStruct((4, _N, 32), jnp.bfloat16),
            jax.ShapeDtypeStruct((_N, _H), f32),
            jax.ShapeDtypeStruct((_N, _H), f32),
            jax.ShapeDtypeStruct((_N, 8), f32),
        ],
    )
    featq, x1, xth, rec = t1(part, x, W1, Wl0, Wr0, b0, a0)

    agg = _make_agg()
    featp = jax.lax.bitcast_convert_type(
        featq.reshape(4 * _N, 16, 2), jnp.int32)
    s1q = agg(featp, src2, dst2).reshape(4, _R, 32)

    t2 = pl.pallas_call(
        _t2_body,
        grid=(grid,),
        in_specs=[
            pl.BlockSpec((4, _RB, 32), lambda i: (0, i, 0)),
            pl.BlockSpec((_RB, _H), lambda i: (i, 0)),
            pl.BlockSpec((_RB, _H), lambda i: (i, 0)),
            pl.BlockSpec((_RB, 8), lambda i: (i, 0)),
            _full((_H, _H)),
            _full((_H, _H)),
            _full((_H,)),
            _full((_H,)),
        ],
        out_specs=[pl.BlockSpec((4, _RB, 32), lambda i: (0, i, 0))],
        out_shape=[jax.ShapeDtypeStruct((4, _N, 32), jnp.bfloat16)],
    )
    (feat2q,) = t2(s1q, x1, xth, rec, Wl1, Wr1, b1, a1)

    feat2p = jax.lax.bitcast_convert_type(
        feat2q.reshape(4 * _N, 16, 2), jnp.int32)
    s2q = agg(feat2p, src2, dst2).reshape(4, _R, 32)

    start = jnp.asarray(batch_size, jnp.int32) - _BOUT
    s2q_b = lax.dynamic_slice_in_dim(s2q, start, _BOUT, axis=1)
    feat2q_b = lax.dynamic_slice_in_dim(feat2q, start, _BOUT, axis=1)
    rec_b = lax.dynamic_slice_in_dim(rec, start, _BOUT, axis=0)

    rb3 = 1024
    t3 = pl.pallas_call(
        _t3_body,
        grid=(_BOUT // rb3,),
        in_specs=[
            pl.BlockSpec((4, rb3, 32), lambda i: (0, i, 0)),
            pl.BlockSpec((4, rb3, 32), lambda i: (0, i, 0)),
            pl.BlockSpec((rb3, 8), lambda i: (i, 0)),
            _full((_H, _H)),
            _full((_H, _H)),
            _full((_H,)),
        ],
        out_specs=[pl.BlockSpec((rb3, _H), lambda i: (i, 0))],
        out_shape=[jax.ShapeDtypeStruct((_BOUT, _H), f32)],
    )
    (out,) = t3(s2q_b, feat2q_b, rec_b, Wl2, Wr2, b2)
    return out


# speedup vs baseline: 1.3099x; 1.3099x over previous
import functools

import jax
import jax.numpy as jnp
from jax import lax
from jax.experimental import pallas as pl
from jax.experimental.pallas import tpu as pltpu
from jax.experimental.pallas import tpu_sc as plsc

_N = 50000
_E = 800000
_INF = 4
_H = 128
_BOUT = 4096

_NTILE = 32
_IX = 128
_CH = 640
_NSUB = _CH // _IX
_EPAD = 819200
_EPT_A = _EPAD // 32
_EPT_B = _EPAD // 16
_R = 50176
_ZROWS = _R // 16

def _mesh():
    return plsc.VectorSubcoreMesh(core_axis_name="c", subcore_axis_name="s")


def _zero_stage(rows):
    def zr(i, c2):
        r = i // 2
        l = (i % 2) * 16
        rows[r, pl.ds(l, 16)] = jnp.zeros((16,), jnp.float32)
        return c2

    lax.fori_loop(0, 1024, zr, 0)


def _zero_acc(acc, rows, s):
    _zero_stage(rows)
    zoff = pl.multiple_of(s * _ZROWS, 8)
    for k in range(7):
        pltpu.sync_copy(rows.at[pl.ds(0, 448)],
                        acc.at[pl.ds(zoff + k * 448, 448)])


def _flush_acc(acc, rows, s, out_h, out_base):
    zoff = pl.multiple_of(s * _ZROWS, 8)
    obase = pl.multiple_of(out_base + s * _ZROWS, 8)
    for k in range(7):
        pltpu.sync_copy(acc.at[pl.ds(zoff + k * 448, 448)],
                        rows.at[pl.ds(0, 448)])
        pltpu.sync_copy(rows.at[pl.ds(0, 448)],
                        out_h.at[pl.ds(obase + k * 448, 448)])


def _fire_idx_loads(src_h, dst_h, iv, dv, rb, sem):
    for j in range(_NSUB):
        pltpu.async_copy(src_h.at[rb + j], iv[j], sem)
        pltpu.async_copy(dst_h.at[rb + j], dv[j], sem)


def _drain_idx_loads(src_h, dst_h, iv, dv, sem):
    for j in range(_NSUB):
        pltpu.make_async_copy(src_h.at[0], iv[j], sem).wait()
        pltpu.make_async_copy(dst_h.at[0], dv[j], sem).wait()


def _agg_pass(feat_h, src_h, dst_h, acc, iv, dv, rows, seml, gsems, erow0,
              nch, qoff):
    iva, ivb = iv[:_NSUB], iv[_NSUB:]
    dva, dvb = dv[:_NSUB], dv[_NSUB:]
    maxrb = erow0 + (nch - 1) * _NSUB

    def half(ck, ivx, dvx, ivn, dvn):
        _drain_idx_loads(src_h, dst_h, ivx, dvx, seml)
        rbn = jnp.minimum(erow0 + (ck + 1) * _NSUB, maxrb)
        _fire_idx_loads(src_h, dst_h, ivn, dvn, rbn, seml)
        if qoff is not None:
            for j in range(_NSUB):
                for l in range(_IX // 16):
                    ivx[j][pl.ds(l * 16, 16)] = (
                        ivx[j][pl.ds(l * 16, 16)] + qoff)
        handles = [
            pltpu.async_copy(feat_h.at[ivx[j]],
                             rows.at[pl.ds(j * _IX, _IX)], gsems[j])
            for j in range(_NSUB)
        ]
        for j in range(_NSUB):
            handles[j].wait()
            pltpu.sync_copy(rows.at[pl.ds(j * _IX, _IX)],
                            acc.at[dvx[j]], add=True)

    _fire_idx_loads(src_h, dst_h, iva, dva, erow0, seml)

    def pair(k, carry):
        half(2 * k, iva, dva, ivb, dvb)
        half(2 * k + 1, ivb, dvb, iva, dva)
        return carry

    lax.fori_loop(0, nch // 2, pair, 0)
    _drain_idx_loads(src_h, dst_h, iva, dva, seml)


def _sc_layer0(xpad_h, src_h, dst_h, out_h, acc,
               d0, d1, d2, d3, d4, d5, d6, d7, d8, d9,
               i0, i1, i2, i3, i4, i5, i6, i7, i8, i9,
               rows, seml, g0, g1, g2, g3, g4):
    c = lax.axis_index("c")
    s = lax.axis_index("s")
    wid = c * 16 + s
    _zero_acc(acc, rows, s)
    plsc.subcore_barrier()
    _agg_pass(xpad_h, src_h, dst_h, acc,
              [i0, i1, i2, i3, i4, i5, i6, i7, i8, i9],
              [d0, d1, d2, d3, d4, d5, d6, d7, d8, d9],
              rows, seml, [g0, g1, g2, g3, g4],
              wid * (_EPT_A // _IX), _EPT_A // _CH, None)
    plsc.subcore_barrier()
    _flush_acc(acc, rows, s, out_h, c * _R)


def _sc_agg(feat_h, src_h, dst_h, out_h, acc,
            d0, d1, d2, d3, d4, d5, d6, d7, d8, d9,
            i0, i1, i2, i3, i4, i5, i6, i7, i8, i9,
            rows, seml, g0, g1, g2, g3, g4):
    c = lax.axis_index("c")
    s = lax.axis_index("s")

    for p in range(2):
        q = c * 2 + p
        qoff = q * _N
        _zero_acc(acc, rows, s)
        plsc.subcore_barrier()
        _agg_pass(feat_h, src_h, dst_h, acc,
                  [i0, i1, i2, i3, i4, i5, i6, i7, i8, i9],
                  [d0, d1, d2, d3, d4, d5, d6, d7, d8, d9],
                  rows, seml, [g0, g1, g2, g3, g4],
                  s * (_EPT_B // _IX), _EPT_B // _CH, qoff)
        plsc.subcore_barrier()
        _flush_acc(acc, rows, s, out_h, q * _R)
        plsc.subcore_barrier()


def _idx_scratch():
    return [pltpu.VMEM((_IX,), jnp.int32) for _ in range(4 * _NSUB)]


def _make_layer0():
    return pl.kernel(
        _sc_layer0,
        out_type=jax.ShapeDtypeStruct((2 * _R, 32), jnp.float32),
        mesh=_mesh(),
        compiler_params=pltpu.CompilerParams(use_tc_tiling_on_sc=False),
        scratch_types=[
            pltpu.VMEM_SHARED((_R, 32), jnp.float32),
        ] + _idx_scratch() + [
            pltpu.VMEM((_CH, 32), jnp.float32),
            pltpu.SemaphoreType.DMA,
            pltpu.SemaphoreType.DMA,
            pltpu.SemaphoreType.DMA,
            pltpu.SemaphoreType.DMA,
            pltpu.SemaphoreType.DMA,
            pltpu.SemaphoreType.DMA,
        ],
    )


def _make_agg():
    return pl.kernel(
        _sc_agg,
        out_type=jax.ShapeDtypeStruct((4 * _R, 32), jnp.float32),
        mesh=_mesh(),
        compiler_params=pltpu.CompilerParams(use_tc_tiling_on_sc=False),
        scratch_types=[
            pltpu.VMEM_SHARED((_R, 32), jnp.float32),
        ] + _idx_scratch() + [
            pltpu.VMEM((_CH, 32), jnp.float32),
            pltpu.SemaphoreType.DMA,
            pltpu.SemaphoreType.DMA,
            pltpu.SemaphoreType.DMA,
            pltpu.SemaphoreType.DMA,
            pltpu.SemaphoreType.DMA,
            pltpu.SemaphoreType.DMA,
        ],
    )


_RB = 2000


def _prelu(v, a):
    return jnp.where(v >= 0, v, a * v)


def _t1_body(part_ref, x_ref, w1_ref, wl0_ref, wr0_ref, b0_ref, a0_ref,
             featq_ref, x1_ref, xth_ref, rec_ref):
    pr = part_ref[...]
    s0 = pr[0, :, 0:4] + pr[1, :, 0:4]
    deg = pr[0, :, 4:5] + pr[1, :, 4:5]
    rec = 1.0 / jnp.maximum(deg, 1.0)
    x = x_ref[...]
    mean0 = s0 * rec
    x1 = _prelu(
        jax.lax.dot_general(mean0, wl0_ref[...], (((1,), (0,)), ((), ())),
                            preferred_element_type=jnp.float32)
        + jax.lax.dot_general(x, wr0_ref[...], (((1,), (0,)), ((), ())),
                              preferred_element_type=jnp.float32)
        + b0_ref[...][None, :],
        a0_ref[...][None, :])
    xth = jax.lax.dot_general(x, w1_ref[...], (((1,), (1,)), ((), ())),
                              preferred_element_type=jnp.float32)
    feat1 = x1 + xth
    for qq in range(4):
        featq_ref[qq] = feat1[:, 32 * qq:32 * (qq + 1)]
    x1_ref[...] = x1
    xth_ref[...] = xth
    rec_ref[...] = jnp.broadcast_to(rec, (rec.shape[0], 8))


def _t2_body(s1q_ref, x1_ref, xth_ref, rec_ref, wl1_ref, wr1_ref,
             b1_ref, a1_ref, feat2q_ref):
    s1 = jnp.concatenate([s1q_ref[qq] for qq in range(4)], axis=1)
    feat1 = x1_ref[...] + xth_ref[...]
    rec = rec_ref[...][:, 0:1]
    mean1 = s1 * rec
    x2 = _prelu(
        jax.lax.dot_general(mean1, wl1_ref[...], (((1,), (0,)), ((), ())),
                            preferred_element_type=jnp.float32)
        + jax.lax.dot_general(feat1, wr1_ref[...], (((1,), (0,)), ((), ())),
                              preferred_element_type=jnp.float32)
        + b1_ref[...][None, :],
        a1_ref[...][None, :])
    h2 = x1_ref[...] + x2
    feat2 = h2 + xth_ref[...]
    for qq in range(4):
        feat2q_ref[qq] = feat2[:, 32 * qq:32 * (qq + 1)]


def _t3_body(s2q_ref, feat2q_ref, rec_ref, wl2_ref, wr2_ref, b2_ref, out_ref):
    s2 = jnp.concatenate([s2q_ref[qq] for qq in range(4)], axis=1)
    feat2 = jnp.concatenate([feat2q_ref[qq] for qq in range(4)],
                            axis=1).astype(jnp.float32)
    rec = rec_ref[...][:, 0:1]
    mean2 = s2 * rec
    out_ref[...] = (
        jax.lax.dot_general(mean2, wl2_ref[...], (((1,), (0,)), ((), ())),
                            preferred_element_type=jnp.float32)
        + jax.lax.dot_general(feat2, wr2_ref[...], (((1,), (0,)), ((), ())),
                              preferred_element_type=jnp.float32)
        + b2_ref[...][None, :])


def _full(shape):
    return pl.BlockSpec(shape, lambda i: tuple(0 for _ in shape))


def kernel(x, edge_index, batch_size, W1, Wl0, Wr0, b0, Wl1, Wr1, b1, Wl2,
           Wr2, b2, a0, a1):
    f32 = jnp.float32
    x = x.astype(f32)
    src = edge_index[0].astype(jnp.int32)
    dst = edge_index[1].astype(jnp.int32)
    npad = _EPAD - _E
    src_p = jnp.concatenate([src, jnp.zeros((npad,), jnp.int32)])
    dst_p = jnp.concatenate([dst, jnp.full((npad,), _N, jnp.int32)])
    src2 = src_p.reshape(_EPAD // _IX, _IX)
    dst2 = dst_p.reshape(_EPAD // _IX, _IX)

    xpad = jnp.concatenate(
        [x, jnp.ones((_N, 1), f32), jnp.zeros((_N, 27), f32)], axis=1)

    part = _make_layer0()(xpad, src2, dst2).reshape(2, _R, 32)

    grid = _N // _RB
    t1 = pl.pallas_call(
        _t1_body,
        grid=(grid,),
        in_specs=[
            pl.BlockSpec((2, _RB, 32), lambda i: (0, i, 0)),
            pl.BlockSpec((_RB, 4), lambda i: (i, 0)),
            _full((_H, _INF)),
            _full((_INF, _H)),
            _full((_INF, _H)),
            _full((_H,)),
            _full((_H,)),
        ],
        out_specs=[
            pl.BlockSpec((4, _RB, 32), lambda i: (0, i, 0)),
            pl.BlockSpec((_RB, _H), lambda i: (i, 0)),
            pl.BlockSpec((_RB, _H), lambda i: (i, 0)),
            pl.BlockSpec((_RB, 8), lambda i: (i, 0)),
        ],
        out_shape=[
            jax.ShapeDtypeStruct((4, _N, 32), f32),
            jax.ShapeDtypeStruct((_N, _H), f32),
            jax.ShapeDtypeStruct((_N, _H), f32),
            jax.ShapeDtypeStruct((_N, 8), f32),
        ],
    )
    featq, x1, xth, rec = t1(part, x, W1, Wl0, Wr0, b0, a0)

    agg = _make_agg()
    s1q = agg(featq.reshape(4 * _N, 32), src2, dst2).reshape(4, _R, 32)

    t2 = pl.pallas_call(
        _t2_body,
        grid=(grid,),
        in_specs=[
            pl.BlockSpec((4, _RB, 32), lambda i: (0, i, 0)),
            pl.BlockSpec((_RB, _H), lambda i: (i, 0)),
            pl.BlockSpec((_RB, _H), lambda i: (i, 0)),
            pl.BlockSpec((_RB, 8), lambda i: (i, 0)),
            _full((_H, _H)),
            _full((_H, _H)),
            _full((_H,)),
            _full((_H,)),
        ],
        out_specs=[pl.BlockSpec((4, _RB, 32), lambda i: (0, i, 0))],
        out_shape=[jax.ShapeDtypeStruct((4, _N, 32), f32)],
    )
    (feat2q,) = t2(s1q, x1, xth, rec, Wl1, Wr1, b1, a1)

    s2q = agg(feat2q.reshape(4 * _N, 32), src2, dst2).reshape(4, _R, 32)

    start = jnp.asarray(batch_size, jnp.int32) - _BOUT
    s2q_b = lax.dynamic_slice_in_dim(s2q, start, _BOUT, axis=1)
    feat2q_b = lax.dynamic_slice_in_dim(feat2q, start, _BOUT, axis=1)
    rec_b = lax.dynamic_slice_in_dim(rec, start, _BOUT, axis=0)

    rb3 = 1024
    t3 = pl.pallas_call(
        _t3_body,
        grid=(_BOUT // rb3,),
        in_specs=[
            pl.BlockSpec((4, rb3, 32), lambda i: (0, i, 0)),
            pl.BlockSpec((4, rb3, 32), lambda i: (0, i, 0)),
            pl.BlockSpec((rb3, 8), lambda i: (i, 0)),
            _full((_H, _H)),
            _full((_H, _H)),
            _full((_H,)),
        ],
        out_specs=[pl.BlockSpec((rb3, _H), lambda i: (i, 0))],
        out_shape=[jax.ShapeDtypeStruct((_BOUT, _H), f32)],
    )
    (out,) = t3(s2q_b, feat2q_b, rec_b, Wl2, Wr2, b2)
    return out


# speedup vs baseline: 1.3101x; 1.0002x over previous
import jax
import jax.numpy as jnp
from jax import lax
from jax.experimental import pallas as pl
from jax.experimental.pallas import tpu as pltpu
from jax.experimental.pallas import tpu_sc as plsc

_N = 50000
_E = 800000
_INF = 4
_H = 128
_BOUT = 4096

_NTILE = 32
_IX = 128
_CH = 640
_NSUB = _CH // _IX
_EPAD = 819200
_EPT_A = _EPAD // 32
_EPT_B = _EPAD // 16
_R = 50176
_ZROWS = _R // 16

def _mesh():
    return plsc.VectorSubcoreMesh(core_axis_name="c", subcore_axis_name="s")


def _zero_stage(rows):
    def zr(i, c2):
        r = i // 2
        l = (i % 2) * 16
        rows[r, pl.ds(l, 16)] = jnp.zeros((16,), jnp.float32)
        return c2

    lax.fori_loop(0, 1024, zr, 0)


def _zero_acc(acc, rows, s):
    _zero_stage(rows)
    zoff = pl.multiple_of(s * _ZROWS, 8)
    for k in range(7):
        pltpu.sync_copy(rows.at[pl.ds(0, 448)],
                        acc.at[pl.ds(zoff + k * 448, 448)])


def _flush_acc(acc, rows, s, out_h, out_base):
    zoff = pl.multiple_of(s * _ZROWS, 8)
    obase = pl.multiple_of(out_base + s * _ZROWS, 8)
    for k in range(7):
        pltpu.sync_copy(acc.at[pl.ds(zoff + k * 448, 448)],
                        rows.at[pl.ds(0, 448)])
        pltpu.sync_copy(rows.at[pl.ds(0, 448)],
                        out_h.at[pl.ds(obase + k * 448, 448)])


def _fire_idx_loads(src_h, dst_h, iv, dv, rb, sem):
    for j in range(_NSUB):
        pltpu.async_copy(src_h.at[rb + j], iv[j], sem)
        pltpu.async_copy(dst_h.at[rb + j], dv[j], sem)


def _drain_idx_loads(src_h, dst_h, iv, dv, sem):
    for j in range(_NSUB):
        pltpu.make_async_copy(src_h.at[0], iv[j], sem).wait()
        pltpu.make_async_copy(dst_h.at[0], dv[j], sem).wait()


def _agg_pass(feat_h, src_h, dst_h, acc, iv, dv, rows, seml, gsems, erow0,
              nch, qoff):
    iva, ivb = iv[:_NSUB], iv[_NSUB:]
    dva, dvb = dv[:_NSUB], dv[_NSUB:]
    maxrb = erow0 + (nch - 1) * _NSUB

    def half(ck, ivx, dvx, ivn, dvn):
        _drain_idx_loads(src_h, dst_h, ivx, dvx, seml)
        rbn = jnp.minimum(erow0 + (ck + 1) * _NSUB, maxrb)
        _fire_idx_loads(src_h, dst_h, ivn, dvn, rbn, seml)
        if qoff is not None:
            for j in range(_NSUB):
                for l in range(_IX // 16):
                    ivx[j][pl.ds(l * 16, 16)] = (
                        ivx[j][pl.ds(l * 16, 16)] + qoff)
        handles = [
            pltpu.async_copy(feat_h.at[ivx[j]],
                             rows.at[pl.ds(j * _IX, _IX)], gsems[j])
            for j in range(_NSUB)
        ]
        for j in range(_NSUB):
            handles[j].wait()
            pltpu.sync_copy(rows.at[pl.ds(j * _IX, _IX)],
                            acc.at[dvx[j]], add=True)

    _fire_idx_loads(src_h, dst_h, iva, dva, erow0, seml)

    def pair(k, carry):
        half(2 * k, iva, dva, ivb, dvb)
        half(2 * k + 1, ivb, dvb, iva, dva)
        return carry

    lax.fori_loop(0, nch // 2, pair, 0)
    _drain_idx_loads(src_h, dst_h, iva, dva, seml)


def _sc_layer0(xpad_h, src_h, dst_h, out_h, acc,
               d0, d1, d2, d3, d4, d5, d6, d7, d8, d9,
               i0, i1, i2, i3, i4, i5, i6, i7, i8, i9,
               rows, seml, g0, g1, g2, g3, g4):
    c = lax.axis_index("c")
    s = lax.axis_index("s")
    wid = c * 16 + s
    _zero_acc(acc, rows, s)
    plsc.subcore_barrier()
    _agg_pass(xpad_h, src_h, dst_h, acc,
              [i0, i1, i2, i3, i4, i5, i6, i7, i8, i9],
              [d0, d1, d2, d3, d4, d5, d6, d7, d8, d9],
              rows, seml, [g0, g1, g2, g3, g4],
              wid * (_EPT_A // _IX), _EPT_A // _CH, None)
    plsc.subcore_barrier()
    _flush_acc(acc, rows, s, out_h, c * _R)


def _sc_agg(feat_h, src_h, dst_h, out_h, acc,
            d0, d1, d2, d3, d4, d5, d6, d7, d8, d9,
            i0, i1, i2, i3, i4, i5, i6, i7, i8, i9,
            rows, seml, g0, g1, g2, g3, g4):
    c = lax.axis_index("c")
    s = lax.axis_index("s")

    for p in range(2):
        q = c * 2 + p
        qoff = q * _N
        _zero_acc(acc, rows, s)
        plsc.subcore_barrier()
        _agg_pass(feat_h, src_h, dst_h, acc,
                  [i0, i1, i2, i3, i4, i5, i6, i7, i8, i9],
                  [d0, d1, d2, d3, d4, d5, d6, d7, d8, d9],
                  rows, seml, [g0, g1, g2, g3, g4],
                  s * (_EPT_B // _IX), _EPT_B // _CH, qoff)
        plsc.subcore_barrier()
        _flush_acc(acc, rows, s, out_h, q * _R)
        plsc.subcore_barrier()


def _idx_scratch():
    return [pltpu.VMEM((_IX,), jnp.int32) for _ in range(4 * _NSUB)]


def _make_layer0():
    return pl.kernel(
        _sc_layer0,
        out_type=jax.ShapeDtypeStruct((2 * _R, 32), jnp.float32),
        mesh=_mesh(),
        compiler_params=pltpu.CompilerParams(use_tc_tiling_on_sc=False),
        scratch_types=[
            pltpu.VMEM_SHARED((_R, 32), jnp.float32),
        ] + _idx_scratch() + [
            pltpu.VMEM((_CH, 32), jnp.float32),
            pltpu.SemaphoreType.DMA,
            pltpu.SemaphoreType.DMA,
            pltpu.SemaphoreType.DMA,
            pltpu.SemaphoreType.DMA,
            pltpu.SemaphoreType.DMA,
            pltpu.SemaphoreType.DMA,
        ],
    )


def _make_agg():
    return pl.kernel(
        _sc_agg,
        out_type=jax.ShapeDtypeStruct((4 * _R, 32), jnp.float32),
        mesh=_mesh(),
        compiler_params=pltpu.CompilerParams(use_tc_tiling_on_sc=False),
        scratch_types=[
            pltpu.VMEM_SHARED((_R, 32), jnp.float32),
        ] + _idx_scratch() + [
            pltpu.VMEM((_CH, 32), jnp.float32),
            pltpu.SemaphoreType.DMA,
            pltpu.SemaphoreType.DMA,
            pltpu.SemaphoreType.DMA,
            pltpu.SemaphoreType.DMA,
            pltpu.SemaphoreType.DMA,
            pltpu.SemaphoreType.DMA,
        ],
    )


_RB = 2000


def _prelu(v, a):
    return jnp.where(v >= 0, v, a * v)


def _t1_body(part_ref, x_ref, w1_ref, wl0_ref, wr0_ref, b0_ref, a0_ref,
             featq_ref, x1_ref, xth_ref, rec_ref):
    pr = part_ref[...]
    s0 = pr[0, :, 0:4] + pr[1, :, 0:4]
    deg = pr[0, :, 4:5] + pr[1, :, 4:5]
    rec = 1.0 / jnp.maximum(deg, 1.0)
    x = x_ref[...]
    mean0 = s0 * rec
    x1 = _prelu(
        jax.lax.dot_general(mean0, wl0_ref[...], (((1,), (0,)), ((), ())),
                            preferred_element_type=jnp.float32)
        + jax.lax.dot_general(x, wr0_ref[...], (((1,), (0,)), ((), ())),
                              preferred_element_type=jnp.float32)
        + b0_ref[...][None, :],
        a0_ref[...][None, :])
    xth = jax.lax.dot_general(x, w1_ref[...], (((1,), (1,)), ((), ())),
                              preferred_element_type=jnp.float32)
    feat1 = x1 + xth
    for qq in range(4):
        featq_ref[qq] = feat1[:, 32 * qq:32 * (qq + 1)]
    x1_ref[...] = x1
    xth_ref[...] = xth
    rec_ref[...] = jnp.broadcast_to(rec, (rec.shape[0], 8))


def _t2_body(s1q_ref, x1_ref, xth_ref, rec_ref, wl1_ref, wr1_ref,
             b1_ref, a1_ref, feat2q_ref):
    s1 = jnp.concatenate([s1q_ref[qq] for qq in range(4)], axis=1)
    feat1 = x1_ref[...] + xth_ref[...]
    rec = rec_ref[...][:, 0:1]
    mean1 = s1 * rec
    x2 = _prelu(
        jax.lax.dot_general(mean1, wl1_ref[...], (((1,), (0,)), ((), ())),
                            preferred_element_type=jnp.float32)
        + jax.lax.dot_general(feat1, wr1_ref[...], (((1,), (0,)), ((), ())),
                              preferred_element_type=jnp.float32)
        + b1_ref[...][None, :],
        a1_ref[...][None, :])
    h2 = x1_ref[...] + x2
    feat2 = h2 + xth_ref[...]
    for qq in range(4):
        feat2q_ref[qq] = feat2[:, 32 * qq:32 * (qq + 1)]


def _t3_body(s2q_ref, feat2q_ref, rec_ref, wl2_ref, wr2_ref, b2_ref, out_ref):
    s2 = jnp.concatenate([s2q_ref[qq] for qq in range(4)], axis=1)
    feat2 = jnp.concatenate([feat2q_ref[qq] for qq in range(4)],
                            axis=1).astype(jnp.float32)
    rec = rec_ref[...][:, 0:1]
    mean2 = s2 * rec
    out_ref[...] = (
        jax.lax.dot_general(mean2, wl2_ref[...], (((1,), (0,)), ((), ())),
                            preferred_element_type=jnp.float32)
        + jax.lax.dot_general(feat2, wr2_ref[...], (((1,), (0,)), ((), ())),
                              preferred_element_type=jnp.float32)
        + b2_ref[...][None, :])


def _full(shape):
    return pl.BlockSpec(shape, lambda i: tuple(0 for _ in shape))


def kernel(x, edge_index, batch_size, W1, Wl0, Wr0, b0, Wl1, Wr1, b1, Wl2,
           Wr2, b2, a0, a1):
    f32 = jnp.float32
    x = x.astype(f32)
    src = edge_index[0].astype(jnp.int32)
    dst = edge_index[1].astype(jnp.int32)
    npad = _EPAD - _E
    src_p = jnp.concatenate([src, jnp.zeros((npad,), jnp.int32)])
    dst_p = jnp.concatenate([dst, jnp.full((npad,), _N, jnp.int32)])
    src2 = src_p.reshape(_EPAD // _IX, _IX)
    dst2 = dst_p.reshape(_EPAD // _IX, _IX)

    xpad = jnp.concatenate(
        [x, jnp.ones((_N, 1), f32), jnp.zeros((_N, 27), f32)], axis=1)

    part = _make_layer0()(xpad, src2, dst2).reshape(2, _R, 32)

    grid = _N // _RB
    t1 = pl.pallas_call(
        _t1_body,
        grid=(grid,),
        in_specs=[
            pl.BlockSpec((2, _RB, 32), lambda i: (0, i, 0)),
            pl.BlockSpec((_RB, 4), lambda i: (i, 0)),
            _full((_H, _INF)),
            _full((_INF, _H)),
            _full((_INF, _H)),
            _full((_H,)),
            _full((_H,)),
        ],
        out_specs=[
            pl.BlockSpec((4, _RB, 32), lambda i: (0, i, 0)),
            pl.BlockSpec((_RB, _H), lambda i: (i, 0)),
            pl.BlockSpec((_RB, _H), lambda i: (i, 0)),
            pl.BlockSpec((_RB, 8), lambda i: (i, 0)),
        ],
        out_shape=[
            jax.ShapeDtypeStruct((4, _N, 32), f32),
            jax.ShapeDtypeStruct((_N, _H), f32),
            jax.ShapeDtypeStruct((_N, _H), f32),
            jax.ShapeDtypeStruct((_N, 8), f32),
        ],
    )
    featq, x1, xth, rec = t1(part, x, W1, Wl0, Wr0, b0, a0)

    agg = _make_agg()
    s1q = agg(featq.reshape(4 * _N, 32), src2, dst2).reshape(4, _R, 32)

    t2 = pl.pallas_call(
        _t2_body,
        grid=(grid,),
        in_specs=[
            pl.BlockSpec((4, _RB, 32), lambda i: (0, i, 0)),
            pl.BlockSpec((_RB, _H), lambda i: (i, 0)),
            pl.BlockSpec((_RB, _H), lambda i: (i, 0)),
            pl.BlockSpec((_RB, 8), lambda i: (i, 0)),
            _full((_H, _H)),
            _full((_H, _H)),
            _full((_H,)),
            _full((_H,)),
        ],
        out_specs=[pl.BlockSpec((4, _RB, 32), lambda i: (0, i, 0))],
        out_shape=[jax.ShapeDtypeStruct((4, _N, 32), f32)],
    )
    (feat2q,) = t2(s1q, x1, xth, rec, Wl1, Wr1, b1, a1)

    s2q = agg(feat2q.reshape(4 * _N, 32), src2, dst2).reshape(4, _R, 32)

    start = jnp.asarray(batch_size, jnp.int32) - _BOUT
    s2q_b = lax.dynamic_slice_in_dim(s2q, start, _BOUT, axis=1)
    feat2q_b = lax.dynamic_slice_in_dim(feat2q, start, _BOUT, axis=1)
    rec_b = lax.dynamic_slice_in_dim(rec, start, _BOUT, axis=0)

    rb3 = 1024
    t3 = pl.pallas_call(
        _t3_body,
        grid=(_BOUT // rb3,),
        in_specs=[
            pl.BlockSpec((4, rb3, 32), lambda i: (0, i, 0)),
            pl.BlockSpec((4, rb3, 32), lambda i: (0, i, 0)),
            pl.BlockSpec((rb3, 8), lambda i: (i, 0)),
            _full((_H, _H)),
            _full((_H, _H)),
            _full((_H,)),
        ],
        out_specs=[pl.BlockSpec((rb3, _H), lambda i: (i, 0))],
        out_shape=[jax.ShapeDtypeStruct((_BOUT, _H), f32)],
    )
    (out,) = t3(s2q_b, feat2q_b, rec_b, Wl2, Wr2, b2)
    return out
